# Initial kernel scaffold; baseline (speedup 1.0000x reference)
import jax
import jax.numpy as jnp
from jax.experimental import pallas as pl


def kernel(x, edge_index_2, edge_index_3, pos_2, pos_3, y, params):
    raise NotImplementedError("write your pallas kernel here")


# speedup vs baseline: 3.5869x; 3.5869x over previous
import functools
import math

import jax
import jax.numpy as jnp
from jax import lax
from jax.experimental import pallas as pl
from jax.experimental.pallas import tpu as pltpu
from jax.experimental.pallas import tpu_sc as plsc

HIDDEN = 64
GROWTH = 32
IN_FEAT = 128
DIM = 3
N2 = 10000
N3 = 50000
E2 = 320000
E3 = 800000

NC = 2
NS = 16
NW = NC * NS

BR = 400
N2P = 10112
N3P = 50048
E2P = 327680
E3P = 819200
BQ = 512
NQP = 51200
RCH = 2000


def _mm_call(xs, ws, bias_row, n_rows):
    k = len(xs)
    M = ws[0].shape[1]
    b_arr = jnp.zeros((8, M), jnp.float32).at[0].set(bias_row)

    def body(*refs):
        xr = refs[:k]
        wr = refs[k:2 * k]
        br = refs[2 * k]
        o = refs[2 * k + 1]
        acc = jnp.dot(xr[0][...], wr[0][...], preferred_element_type=jnp.float32)
        for i in range(1, k):
            acc = acc + jnp.dot(xr[i][...], wr[i][...],
                                preferred_element_type=jnp.float32)
        o[...] = acc + br[0:1, :]

    in_specs = (
        [pl.BlockSpec((BR, x.shape[1]), lambda i: (i, 0)) for x in xs]
        + [pl.BlockSpec(w.shape, lambda i: (0, 0)) for w in ws]
        + [pl.BlockSpec((8, M), lambda i: (0, 0))]
    )
    return pl.pallas_call(
        body,
        grid=(n_rows // BR,),
        in_specs=in_specs,
        out_specs=pl.BlockSpec((BR, M), lambda i: (i, 0)),
        out_shape=jax.ShapeDtypeStruct((n_rows, M), jnp.float32),
    )(*xs, *ws, b_arr)


def _combine_stats(parts_list, r, n_rows):
    widths = [p.shape[2] for p in parts_list]
    F = sum(widths)
    npart = len(parts_list)
    has_r = r is not None

    def body(*refs):
        prefs = refs[:npart]
        off = npart
        if has_r:
            rref = refs[off]
            off += 1
        h_ref, s_ref = refs[off], refs[off + 1]
        segs = []
        for pr in prefs:
            p = pr[...]
            segs.append(p[0] + p[1])
        h = segs[0] if len(segs) == 1 else jnp.concatenate(segs, axis=1)
        if has_r:
            h = h + rref[...]
        h_ref[...] = h

        @pl.when(pl.program_id(0) == 0)
        def _():
            s_ref[...] = jnp.zeros_like(s_ref)

        s_ref[0:1, :] = s_ref[0:1, :] + jnp.sum(h, axis=0, keepdims=True)
        s_ref[1:2, :] = s_ref[1:2, :] + jnp.sum(h * h, axis=0, keepdims=True)

    in_specs = [
        pl.BlockSpec((2, BR, w), lambda i: (0, i, 0)) for w in widths
    ]
    args = list(parts_list)
    if has_r:
        in_specs.append(pl.BlockSpec((BR, F), lambda i: (i, 0)))
        args.append(r)
    return pl.pallas_call(
        body,
        grid=(n_rows // BR,),
        in_specs=in_specs,
        out_specs=[
            pl.BlockSpec((BR, F), lambda i: (i, 0)),
            pl.BlockSpec((8, F), lambda i: (0, 0)),
        ],
        out_shape=[
            jax.ShapeDtypeStruct((n_rows, F), jnp.float32),
            jax.ShapeDtypeStruct((8, F), jnp.float32),
        ],
    )(*args)


def _bn_apply(h, stats, gamma, beta, n_rows, elu, residuals, alpha=0.2):
    F = h.shape[1]
    gb = jnp.zeros((8, F), jnp.float32).at[0].set(gamma).at[1].set(beta)
    nres = len(residuals)
    inv_n = 1.0 / float(n_rows)

    def body(*refs):
        h_ref, s_ref, gb_ref = refs[0], refs[1], refs[2]
        res_refs = refs[3:3 + nres]
        out_refs = refs[3 + nres:]
        mu = s_ref[0:1, :] * inv_n
        var = s_ref[1:2, :] * inv_n - mu * mu
        a = (h_ref[...] - mu) * lax.rsqrt(var + 1e-5) * gb_ref[0:1, :] \
            + gb_ref[1:2, :]
        if elu:
            a = jnp.where(a > 0, a, jnp.exp(a) - 1.0)
        if nres == 0:
            out_refs[0][...] = a
        else:
            cur = a
            for j in range(nres):
                cur = res_refs[j][...] + alpha * cur
                out_refs[j][...] = cur

    n_out = max(nres, 1)
    in_specs = [
        pl.BlockSpec((BR, F), lambda i: (i, 0)),
        pl.BlockSpec((8, F), lambda i: (0, 0)),
        pl.BlockSpec((8, F), lambda i: (0, 0)),
    ] + [pl.BlockSpec((BR, F), lambda i: (i, 0)) for _ in range(nres)]
    outs = pl.pallas_call(
        body,
        grid=(n_rows // BR,),
        in_specs=in_specs,
        out_specs=[pl.BlockSpec((BR, F), lambda i: (i, 0))
                   for _ in range(n_out)],
        out_shape=[jax.ShapeDtypeStruct((n_rows, F), jnp.float32)
                   for _ in range(n_out)],
    )(h, stats, gb, *residuals)
    return outs if nres > 0 else outs[0]


def _knn_argmin(pxp, pyT):
    def body(px_ref, py_ref, o_ref):
        best = jnp.full((1, BQ), jnp.inf, dtype=jnp.float32)
        bidx = jnp.zeros((1, BQ), dtype=jnp.int32)
        for c in range(N2 // RCH):
            d2 = None
            for d in range(3):
                diff = py_ref[d:d + 1, :] - px_ref[c * RCH:(c + 1) * RCH, d:d + 1]
                sq = diff * diff
                d2 = sq if d2 is None else d2 + sq
            m = jnp.min(d2, axis=0, keepdims=True)
            ids = lax.broadcasted_iota(jnp.int32, (RCH, BQ), 0) + (c * RCH)
            am = jnp.min(jnp.where(d2 == m, ids, jnp.int32(2 ** 30)),
                         axis=0, keepdims=True)
            upd = m < best
            bidx = jnp.where(upd, am, bidx)
            best = jnp.where(upd, m, best)
        o_ref[0] = bidx

    return pl.pallas_call(
        body,
        grid=(NQP // BQ,),
        in_specs=[
            pl.BlockSpec((N2, 8), lambda i: (0, 0)),
            pl.BlockSpec((8, BQ), lambda i: (0, i)),
        ],
        out_specs=pl.BlockSpec((1, 1, BQ), lambda i: (i, 0, 0)),
        out_shape=jax.ShapeDtypeStruct((NQP // BQ, 1, BQ), jnp.int32),
    )(pxp, pyT)


def _sc_segsum(h, src2d, dst2d, n_pad, d_feat, epw_rows):
    rows_per_sub = n_pad // NS
    n_chunks = epw_rows // 8
    mesh = plsc.VectorSubcoreMesh(core_axis_name="c", subcore_axis_name="s")

    @functools.partial(
        pl.kernel,
        out_type=jax.ShapeDtypeStruct((NC * n_pad, d_feat), jnp.float32),
        mesh=mesh,
        scratch_types=[
            pltpu.VMEM((8, 128), jnp.int32),
            pltpu.VMEM((8, 128), jnp.int32),
            pltpu.VMEM((1024, d_feat), jnp.float32),
            pltpu.VMEM((128, d_feat), jnp.float32),
            pltpu.VMEM_SHARED((n_pad, d_feat), jnp.float32),
            pltpu.SemaphoreType.DMA,
        ],
        compiler_params=pltpu.CompilerParams(use_tc_tiling_on_sc=False),
    )
    def k(h_hbm, src_hbm, dst_hbm, out_hbm, sidx_v, didx_v, rows_v, zbuf_v,
          acc, sem):
        cid = lax.axis_index("c")
        sid = lax.axis_index("s")
        wid = cid * NS + sid

        def zrow(i, carry):
            for cc in range(d_feat // 16):
                zbuf_v[i, pl.ds(cc * 16, 16)] = jnp.zeros((16,), jnp.float32)
            return carry

        lax.fori_loop(0, 128, zrow, 0)

        r0 = sid * rows_per_sub
        zfull, zrem = divmod(rows_per_sub, 128)
        for kk in range(zfull):
            pltpu.sync_copy(zbuf_v, acc.at[pl.ds(r0 + kk * 128, 128)])
        if zrem:
            pltpu.sync_copy(zbuf_v.at[pl.ds(0, zrem)],
                            acc.at[pl.ds(r0 + zfull * 128, zrem)])
        plsc.subcore_barrier()

        row_base = wid * epw_rows

        def chunk(kk, carry):
            rb = row_base + kk * 8
            pltpu.sync_copy(src_hbm.at[pl.ds(rb, 8)], sidx_v)
            pltpu.sync_copy(dst_hbm.at[pl.ds(rb, 8)], didx_v)
            cps = [
                pltpu.async_copy(h_hbm.at[sidx_v.at[j]],
                                 rows_v.at[pl.ds(j * 128, 128)], sem)
                for j in range(8)
            ]
            for cp in cps:
                cp.wait()
            for j in range(8):
                pltpu.sync_copy(rows_v.at[pl.ds(j * 128, 128)],
                                acc.at[didx_v.at[j]], add=True)
            return carry

        lax.fori_loop(0, n_chunks, chunk, 0)
        plsc.subcore_barrier()

        obase = cid * n_pad + r0
        wfull, wrem = divmod(rows_per_sub, 1024)
        for kk in range(wfull):
            pltpu.sync_copy(acc.at[pl.ds(r0 + kk * 1024, 1024)], rows_v)
            pltpu.sync_copy(rows_v, out_hbm.at[pl.ds(obase + kk * 1024, 1024)])
        if wrem:
            pltpu.sync_copy(acc.at[pl.ds(r0 + wfull * 1024, wrem)],
                            rows_v.at[pl.ds(0, wrem)])
            pltpu.sync_copy(rows_v.at[pl.ds(0, wrem)],
                            out_hbm.at[pl.ds(obase + wfull * 1024, wrem)])

    return k(h, src2d, dst2d).reshape(NC, n_pad, d_feat)


def _sc_gather(table, idx2d, b_pad):
    d_feat = table.shape[1]
    per_w = b_pad // NW
    rows_per_w = per_w // 128
    mesh = plsc.VectorSubcoreMesh(core_axis_name="c", subcore_axis_name="s")

    @functools.partial(
        pl.kernel,
        out_type=jax.ShapeDtypeStruct((b_pad, d_feat), jnp.float32),
        mesh=mesh,
        scratch_types=[
            pltpu.VMEM((8, 128), jnp.int32),
            pltpu.VMEM((512, d_feat), jnp.float32),
            pltpu.SemaphoreType.DMA,
        ],
        compiler_params=pltpu.CompilerParams(use_tc_tiling_on_sc=False),
    )
    def k(tab_hbm, idx_hbm, out_hbm, iidx_v, rows_v, sem):
        cid = lax.axis_index("c")
        sid = lax.axis_index("s")
        wid = cid * NS + sid
        for kk in range(rows_per_w // 8):
            pltpu.sync_copy(idx_hbm.at[pl.ds(wid * rows_per_w + kk * 8, 8)],
                            iidx_v)
            for half in range(2):
                cps = [
                    pltpu.async_copy(tab_hbm.at[iidx_v.at[half * 4 + j]],
                                     rows_v.at[pl.ds(j * 128, 128)], sem)
                    for j in range(4)
                ]
                for cp in cps:
                    cp.wait()
                pltpu.sync_copy(
                    rows_v,
                    out_hbm.at[pl.ds(wid * per_w + kk * 1024 + half * 512,
                                     512)])

    return k(table, idx2d)


def _final_conv5(parts_list, x, wrel, wroot, bias_row):
    b_arr = jnp.zeros((8, IN_FEAT), jnp.float32).at[0].set(bias_row)
    npart = len(parts_list)
    widths = [p.shape[2] for p in parts_list]

    def body(*refs):
        prefs = refs[:npart]
        x_ref, wrel_ref, wroot_ref, b_ref, o_ref = refs[npart:]
        segs = []
        for pr in prefs:
            p = pr[...]
            segs.append(p[0] + p[1])
        agg = jnp.concatenate(segs, axis=1)
        o_ref[...] = (
            jnp.dot(agg, wrel_ref[...], preferred_element_type=jnp.float32)
            + jnp.dot(x_ref[...], wroot_ref[...],
                      preferred_element_type=jnp.float32)
            + b_ref[0:1, :]
        )

    return pl.pallas_call(
        body,
        grid=(N3 // BR,),
        in_specs=[pl.BlockSpec((2, BR, w), lambda i: (0, i, 0))
                  for w in widths] + [
            pl.BlockSpec((BR, HIDDEN), lambda i: (i, 0)),
            pl.BlockSpec((HIDDEN, IN_FEAT), lambda i: (0, 0)),
            pl.BlockSpec((HIDDEN, IN_FEAT), lambda i: (0, 0)),
            pl.BlockSpec((8, IN_FEAT), lambda i: (0, 0)),
        ],
        out_specs=pl.BlockSpec((BR, IN_FEAT), lambda i: (i, 0)),
        out_shape=jax.ShapeDtypeStruct((N3, IN_FEAT), jnp.float32),
    )(*parts_list, x, wrel, wroot, b_arr)


def _onera_transform(pos):
    p0 = pos[:, 0] - math.tan(math.pi / 6) * pos[:, 1]
    pos = jnp.concatenate([p0[:, None], pos[:, 1:]], axis=1)
    return pos * (1 + (1 / 0.56 - 1) * (pos[:, 1:2] / 1.1963))


def _pad_edges(ei, ep, dummy):
    e = ei.shape[1]
    src = jnp.concatenate([ei[0], jnp.zeros((ep - e,), jnp.int32)])
    dst = jnp.concatenate([ei[1], jnp.full((ep - e,), dummy, jnp.int32)])
    return src.reshape(-1, 128), dst.reshape(-1, 128)


def _split_w(p, fins):
    wrel, wroot, b = p['Wrel'], p['Wroot'], p['b']
    fout = wrel.shape[1]
    ws = []
    off = 0
    for fin in fins:
        ws.append(jnp.concatenate([wrel[off:off + fin], wroot[off:off + fin]],
                                  axis=1))
        off += fin
    bias_row = jnp.concatenate([jnp.zeros((fout,), jnp.float32), b])
    return ws, bias_row, fout


def _conv_ei2(p, xs, src2, dst2):
    fins = [a.shape[1] for a in xs]
    ws, bias_row, fout = _split_w(p, fins)
    pr = _mm_call(xs, ws, bias_row, N2)
    P, R = pr[:, :fout], pr[:, fout:]
    parts = _sc_segsum(P, src2, dst2, N2P, fout, E2P // 128 // NW)
    return _combine_stats([parts], R, N2)


def _rdb(p, x, src2, dst2, residuals):
    h1, s1 = _conv_ei2(p['conv1'], [x], src2, dst2)
    y1 = _bn_apply(h1, s1, p['bn1']['gamma'], p['bn1']['beta'], N2,
                   elu=True, residuals=[])
    h2, s2 = _conv_ei2(p['conv2'], [x, y1], src2, dst2)
    y2 = _bn_apply(h2, s2, p['bn2']['gamma'], p['bn2']['beta'], N2,
                   elu=True, residuals=[])
    h3, s3 = _conv_ei2(p['conv3'], [x, y1, y2], src2, dst2)
    return _bn_apply(h3, s3, p['bn3']['gamma'], p['bn3']['beta'], N2,
                     elu=True, residuals=residuals)


def _erdb(p, x, src2, dst2, extra_res):
    outs1 = _rdb(p['rdb1'], x, src2, dst2, residuals=[x, x])
    in2 = outs1[1]
    return _rdb(p['rdb2'], in2, src2, dst2,
                residuals=[in2, x] + extra_res)


def kernel(x, edge_index_2, edge_index_3, pos_2, pos_3, y, params):
    f32 = jnp.float32
    ei2 = edge_index_2.astype(jnp.int32)
    ei3 = edge_index_3.astype(jnp.int32)
    src2, dst2 = _pad_edges(ei2, E2P, N2)
    src3, dst3 = _pad_edges(ei3, E3P, N3)

    inp = jnp.concatenate([x, pos_2, y * jnp.ones_like(pos_2)], axis=1)
    inp = jnp.pad(inp, ((0, 0), (0, 256 - inp.shape[1])))

    p1 = params['conv1']
    w1 = jnp.concatenate([p1['Wrel'], p1['Wroot']], axis=1)
    w1 = jnp.pad(w1, ((0, 256 - w1.shape[0]), (0, 0)))
    b1 = jnp.concatenate([jnp.zeros((HIDDEN,), f32), p1['b']])
    pr1 = _mm_call([inp], [w1], b1, N2)
    P1, R1 = pr1[:, :HIDDEN], pr1[:, HIDDEN:]
    parts1 = _sc_segsum(P1, src2, dst2, N2P, HIDDEN, E2P // 128 // NW)
    hc1, sc1 = _combine_stats([parts1], R1, N2)
    x1 = _bn_apply(hc1, sc1, params['bn1']['gamma'], params['bn1']['beta'],
                   N2, elu=False, residuals=[])

    x2 = _erdb(params['erdb1'], x1, src2, dst2, extra_res=[])[1]
    x3 = _erdb(params['erdb2'], x2, src2, dst2, extra_res=[x1])[2]

    pxt = _onera_transform(pos_2)
    pyt = _onera_transform(pos_3)
    pxp = jnp.pad(pxt, ((0, 0), (0, 5)))
    pyT = jnp.pad(pyt, ((0, NQP - N3), (0, 5))).T
    idx3 = _knn_argmin(pxp, pyT).reshape(-1)[:N3]

    p3 = params['conv3']
    w3 = jnp.concatenate([p3['Wrel'], p3['Wroot']], axis=1)
    b3 = jnp.concatenate([jnp.zeros((HIDDEN,), f32), p3['b']])
    q = _mm_call([x3], [w3], b3, N2)
    bgp = NW * 2048
    idxp = jnp.zeros((bgp,), jnp.int32).at[:N3].set(idx3).reshape(-1, 128)
    qi = _sc_gather(q, idxp, bgp)

    R3 = qi[:N3, HIDDEN:]
    parts3 = [
        _sc_segsum(qi[:N3, w:w + 16], src3, dst3, N3P, 16, E3P // 128 // NW)
        for w in range(0, HIDDEN, 16)
    ]
    h3, s3 = _combine_stats(parts3, R3, N3)
    xi1 = _bn_apply(h3, s3, params['bn3']['gamma'], params['bn3']['beta'],
                    N3, elu=True, residuals=[])

    p4 = params['conv4']
    w4 = jnp.concatenate([p4['Wrel'], p4['Wroot']], axis=1)
    b4 = jnp.concatenate([jnp.zeros((HIDDEN,), f32), p4['b']])
    pr4 = _mm_call([xi1], [w4], b4, N3)
    parts4 = [
        _sc_segsum(pr4[:, w:w + 16], src3, dst3, N3P, 16, E3P // 128 // NW)
        for w in range(0, HIDDEN, 16)
    ]
    h4, s4 = _combine_stats(parts4, pr4[:, HIDDEN:], N3)
    xi2 = _bn_apply(h4, s4, params['bn4']['gamma'], params['bn4']['beta'],
                    N3, elu=True, residuals=[])

    p5 = params['conv5']
    parts5 = [
        _sc_segsum(xi2[:, w:w + 16], src3, dst3, N3P, 16, E3P // 128 // NW)
        for w in range(0, HIDDEN, 16)
    ]
    return _final_conv5(parts5, xi2, p5['Wrel'], p5['Wroot'], p5['b'])
